# Initial kernel scaffold; baseline (speedup 1.0000x reference)
import jax
import jax.numpy as jnp
from jax.experimental import pallas as pl


def kernel(x, edge_index, W1, b1, gamma1, beta1, W2, b2, gamma2, beta2, W3, b3, gamma3, beta3, p1, p2):
    raise NotImplementedError("write your pallas kernel here")


# speedup vs baseline: 1.1313x; 1.1313x over previous
import functools

import jax
import jax.numpy as jnp
from jax import lax
from jax.experimental import pallas as pl
from jax.experimental.pallas import tpu as pltpu
from jax.experimental.pallas import tpu_sc as plsc

N = 10000
E = 320000
D = 128
K1 = 7500
K2 = 5625

C = 128
NTILES = 32
NB = 79
EP = NTILES * NB * C
NP1 = 10240
NP2 = 7680
K2P = 5632
G1 = NP1 - N
G2 = NP2 - K1
NBC1 = 96
NBC2 = 64


def _mesh():
    return plsc.VectorSubcoreMesh(core_axis_name="c", subcore_axis_name="s")


@functools.lru_cache(maxsize=None)
def _msgpass(np_, nbc):
    rows_t = np_ // NTILES

    @functools.partial(
        pl.kernel,
        out_type=jax.ShapeDtypeStruct((np_, D), jnp.float32),
        mesh=_mesh(),
        scratch_types=[
            pltpu.VMEM((C,), jnp.int32),
            pltpu.VMEM((C,), jnp.int32),
            pltpu.VMEM((C, D), jnp.float32),
            pltpu.VMEM((C,), jnp.float32),
            pltpu.VMEM((C,), jnp.float32),
            pltpu.VMEM((rows_t, D), jnp.float32),
            pltpu.SemaphoreType.DMA,
        ],
    )
    def k(h_hbm, dinv_hbm, src_hbm, dst_hbm, out_hbm,
          srcv, dstv, rows, dsv, ddv, acc, sem):
        cid = lax.axis_index("c")
        sid = lax.axis_index("s")
        wid = cid * 16 + sid
        base = wid * rows_t

        @pl.loop(0, rows_t)
        def _(r):
            for kk in range(D // 16):
                acc[r, pl.ds(kk * 16, 16)] = jnp.zeros((16,), jnp.float32)

        @pl.loop(0, nbc)
        def _(j):
            pltpu.sync_copy(src_hbm.at[wid, j], srcv)
            pltpu.sync_copy(dst_hbm.at[wid, j], dstv)
            pltpu.async_copy(h_hbm.at[srcv], rows, sem).wait()
            pltpu.async_copy(dinv_hbm.at[srcv], dsv, sem).wait()
            pltpu.async_copy(dinv_hbm.at[dstv], ddv, sem).wait()

            @pl.loop(0, C // 16)
            def _(g):
                sl16 = pl.ds(g * 16, 16)
                dv = dstv[sl16] - base
                nv = dsv[sl16] * ddv[sl16]
                for t in range(16):
                    dl = dv[t]
                    nm = nv[t]
                    i = g * 16 + t
                    for kk in range(D // 16):
                        sl = pl.ds(kk * 16, 16)
                        acc[dl, sl] = acc[dl, sl] + rows[i, sl] * nm

        pltpu.sync_copy(acc, out_hbm.at[pl.ds(base, rows_t), :])

    return k


@functools.lru_cache(maxsize=None)
def _deg_build(np_, ep):
    et = ep // NTILES
    nb = et // C
    nr = np_ // 16

    @functools.partial(
        pl.kernel,
        out_type=jax.ShapeDtypeStruct((2 * np_,), jnp.float32),
        mesh=_mesh(),
        scratch_types=[
            pltpu.VMEM((nb, C), jnp.int32),
            pltpu.VMEM((C,), jnp.float32),
            pltpu.VMEM((nr,), jnp.float32),
            pltpu.VMEM_SHARED((np_,), jnp.float32),
        ],
    )
    def k(dst_hbm, out_hbm, dstv, ones_v, zbuf, accum):
        cid = lax.axis_index("c")
        sid = lax.axis_index("s")
        wid = cid * 16 + sid
        for i in range(C // 16):
            ones_v[pl.ds(i * 16, 16)] = jnp.full((16,), 1.0, jnp.float32)
        for i in range(nr // 16):
            zbuf[pl.ds(i * 16, 16)] = jnp.zeros((16,), jnp.float32)
        pltpu.sync_copy(zbuf, accum.at[pl.ds(sid * nr, nr)])
        pltpu.sync_copy(dst_hbm.at[wid], dstv)
        plsc.subcore_barrier()

        @pl.loop(0, nb)
        def _(j):
            pltpu.sync_copy(ones_v, accum.at[dstv.at[j]], add=True)

        plsc.subcore_barrier()
        pltpu.sync_copy(accum.at[pl.ds(sid * nr, nr)], zbuf)
        pltpu.sync_copy(zbuf, out_hbm.at[pl.ds(cid * np_ + sid * nr, nr)])

    return k


@functools.lru_cache(maxsize=None)
def _pool_gather(nt, kp):
    per = kp // NTILES
    chunks = []
    off = 0
    while off < per:
        cs = min(C, per - off)
        chunks.append((off, cs))
        off += cs

    @functools.partial(
        pl.kernel,
        out_type=jax.ShapeDtypeStruct((kp, D), jnp.float32),
        mesh=_mesh(),
        scratch_types=[
            pltpu.VMEM((per,), jnp.int32),
            pltpu.VMEM((C, D), jnp.float32),
            pltpu.SemaphoreType.DMA,
        ],
    )
    def k(tab_hbm, idx_hbm, out_hbm, idxv, rows, sem):
        cid = lax.axis_index("c")
        sid = lax.axis_index("s")
        wid = cid * 16 + sid
        base = wid * per
        pltpu.sync_copy(idx_hbm.at[pl.ds(base, per)], idxv)
        for off, cs in chunks:
            pltpu.async_copy(tab_hbm.at[idxv.at[pl.ds(off, cs)]],
                             rows.at[pl.ds(0, cs)], sem).wait()
            pltpu.sync_copy(rows.at[pl.ds(0, cs)],
                            out_hbm.at[pl.ds(base + off, cs), :])

    return k


_BNF = None


def _mm_body(x_ref, w_ref, ex_ref, o_ref):
    o_ref[...] = jnp.dot(x_ref[...] * ex_ref[...], w_ref[...],
                         preferred_element_type=jnp.float32)


@functools.lru_cache(maxsize=None)
def _mm(np_, br=512):
    grid = (np_ // br,)
    return pl.pallas_call(
        _mm_body,
        grid=grid,
        in_specs=[
            pl.BlockSpec((br, D), lambda i: (i, 0)),
            pl.BlockSpec((D, D), lambda i: (0, 0)),
            pl.BlockSpec((br, 1), lambda i: (i, 0)),
        ],
        out_specs=pl.BlockSpec((br, D), lambda i: (i, 0)),
        out_shape=jax.ShapeDtypeStruct((np_, D), jnp.float32),
    )


def _post(acc, h, dinv_ref, b, ga, be):
    dinv = dinv_ref[...]
    pre = (acc[...] + h[...] * (dinv * dinv)) + b[...]
    bn = pre / jnp.sqrt(jnp.float32(1.0 + 1e-5)) * ga[...] + be[...]
    return jnp.maximum(bn, 0.0)


def _ep_score_body(acc, h, dinv_ref, b, ga, be, p_ref, h_ref, s_ref):
    hh = _post(acc, h, dinv_ref, b, ga, be)
    p = p_ref[...]
    h_ref[...] = hh
    s_ref[...] = jnp.tanh(jnp.dot(hh, p.reshape(D, 1),
                                  preferred_element_type=jnp.float32)
                          / jnp.sqrt(jnp.sum(p * p)))


@functools.lru_cache(maxsize=None)
def _ep_score(np_, br=512):
    grid = (np_ // br,)
    row = lambda i: (i, 0)
    fix = lambda i: (0, 0)
    return pl.pallas_call(
        _ep_score_body,
        grid=grid,
        in_specs=[
            pl.BlockSpec((br, D), row), pl.BlockSpec((br, D), row),
            pl.BlockSpec((br, 1), row),
            pl.BlockSpec((1, D), fix), pl.BlockSpec((1, D), fix),
            pl.BlockSpec((1, D), fix), pl.BlockSpec((1, D), fix),
        ],
        out_specs=[pl.BlockSpec((br, D), row), pl.BlockSpec((br, 1), row)],
        out_shape=[jax.ShapeDtypeStruct((np_, D), jnp.float32),
                   jax.ShapeDtypeStruct((np_, 1), jnp.float32)],
    )


def _ep_mm_body(acc, h, dinv_ref, b, ga, be, w_ref, o_ref):
    hh = _post(acc, h, dinv_ref, b, ga, be)
    o_ref[...] = jnp.dot(hh, w_ref[...], preferred_element_type=jnp.float32)


@functools.lru_cache(maxsize=None)
def _ep_mm(np_, br=512):
    grid = (np_ // br,)
    row = lambda i: (i, 0)
    fix = lambda i: (0, 0)
    return pl.pallas_call(
        _ep_mm_body,
        grid=grid,
        in_specs=[
            pl.BlockSpec((br, D), row), pl.BlockSpec((br, D), row),
            pl.BlockSpec((br, 1), row),
            pl.BlockSpec((1, D), fix), pl.BlockSpec((1, D), fix),
            pl.BlockSpec((1, D), fix), pl.BlockSpec((D, D), fix),
        ],
        out_specs=pl.BlockSpec((br, D), row),
        out_shape=jax.ShapeDtypeStruct((np_, D), jnp.float32),
    )


def _rowscale_body(x_ref, v_ref, o_ref):
    o_ref[...] = x_ref[...] * v_ref[...]


@functools.lru_cache(maxsize=None)
def _rowscale(np_, br=512):
    grid = (np_ // br,)
    return pl.pallas_call(
        _rowscale_body,
        grid=grid,
        in_specs=[pl.BlockSpec((br, D), lambda i: (i, 0)),
                  pl.BlockSpec((br, 1), lambda i: (i, 0))],
        out_specs=pl.BlockSpec((br, D), lambda i: (i, 0)),
        out_shape=jax.ShapeDtypeStruct((np_, D), jnp.float32),
    )


def _bucketize(src, dst, keep, nreal, np_, nbc, deg_sum):
    i32 = jnp.int32
    rows_t = np_ // NTILES
    cap = nbc * C
    e = src.shape[0]
    bucket = jnp.where(keep, dst // rows_t, NTILES).astype(jnp.uint32)
    key = (bucket << 19) | jnp.arange(e, dtype=jnp.uint32)
    skey = jnp.sort(key)
    e_sorted = (skey & jnp.uint32((1 << 19) - 1)).astype(i32)
    b_sorted = (skey >> 19).astype(i32)
    degm = jnp.where(jnp.arange(np_) < nreal, deg_sum, 0.0)
    counts = degm.reshape(NTILES, rows_t).sum(1).astype(i32)
    offsets = jnp.concatenate(
        [jnp.zeros((1,), i32), jnp.cumsum(counts)[:-1], jnp.zeros((1,), i32)])
    rank = jnp.arange(e, dtype=i32) - offsets[b_sorted]
    pos = b_sorted * cap + rank
    base_dst = jnp.repeat(jnp.arange(NTILES, dtype=i32) * rows_t, cap)
    src_pad = jnp.full((NTILES * cap,), np_ - 1, i32).at[pos].set(src[e_sorted])
    dst_pad = base_dst.at[pos].set(dst[e_sorted])
    return src_pad.reshape(NTILES, nbc, C), dst_pad.reshape(NTILES, nbc, C)


def kernel(x, edge_index, W1, b1, gamma1, beta1, W2, b2, gamma2, beta2,
           W3, b3, gamma3, beta3, p1, p2):
    f32 = jnp.float32
    i32 = jnp.int32
    ei = edge_index

    x_pad = jnp.pad(x, ((0, NP1 - N), (0, 0)))
    pad1 = N + (jnp.arange(EP - E, dtype=i32) % G1)
    dst1 = jnp.concatenate([ei[1], pad1]).reshape(NTILES, NB, C)

    degp = _deg_build(NP1, EP)(dst1).reshape(2, NP1)
    deg1 = degp[0] + degp[1]
    dinv1 = jnp.where(deg1 + 1.0 > 0, 1.0 / jnp.sqrt(deg1 + 1.0), 0.0)
    rowmask1 = (jnp.arange(NP1) < N).astype(f32).reshape(NP1, 1)

    H1 = _mm(NP1)(x_pad, W1, rowmask1)
    keep1 = jnp.ones((E,), jnp.bool_)
    s1p, d1p = _bucketize(ei[0], ei[1], keep1, N, NP1, NBC1, deg1)
    acc1 = _msgpass(NP1, NBC1)(H1, dinv1, s1p, d1p)
    h0, sc1 = _ep_score(NP1)(
        acc1, H1, dinv1.reshape(NP1, 1),
        b1.reshape(1, D), gamma1.reshape(1, D), beta1.reshape(1, D),
        p1.reshape(1, D))

    score1 = sc1[:N, 0]
    vals1, perm1 = lax.top_k(score1, K1)
    newid = jnp.full((N,), -1, i32).at[perm1].set(jnp.arange(K1, dtype=i32))
    s2 = newid[ei[0]]
    d2 = newid[ei[1]]
    keep = (s2 >= 0) & (d2 >= 0)
    spread = K1 + (jnp.arange(E, dtype=i32) % G2)
    d2s = jnp.where(keep, d2, spread)
    pad2 = K1 + (jnp.arange(EP - E, dtype=i32) % G2)
    dst2 = jnp.concatenate([d2s, pad2]).reshape(NTILES, NB, C)

    perm1_pad = jnp.concatenate([perm1, jnp.arange(NP2 - K1, dtype=i32)])
    h1_raw = _pool_gather(NP1, NP2)(h0, perm1_pad)

    degp2 = _deg_build(NP2, EP)(dst2).reshape(2, NP2)
    deg2 = degp2[0] + degp2[1]
    dinv2 = jnp.where(deg2 + 1.0 > 0, 1.0 / jnp.sqrt(deg2 + 1.0), 0.0)
    extra2 = jnp.pad(vals1, (0, NP2 - K1)).reshape(NP2, 1)

    s2p, d2p = _bucketize(s2, d2, keep, K1, NP2, NBC2, deg2)
    H2 = _mm(NP2)(h1_raw, W2, extra2)
    acc2 = _msgpass(NP2, NBC2)(H2, dinv2, s2p, d2p)
    H3 = _ep_mm(NP2)(
        acc2, H2, dinv2.reshape(NP2, 1),
        b2.reshape(1, D), gamma2.reshape(1, D), beta2.reshape(1, D), W3)
    acc3 = _msgpass(NP2, NBC2)(H3, dinv2, s2p, d2p)
    h3, sc2 = _ep_score(NP2)(
        acc3, H3, dinv2.reshape(NP2, 1),
        b3.reshape(1, D), gamma3.reshape(1, D), beta3.reshape(1, D),
        p2.reshape(1, D))

    score2 = sc2[:K1, 0]
    vals2, perm2 = lax.top_k(score2, K2)
    perm2_pad = jnp.concatenate([perm2, jnp.zeros((K2P - K2,), i32)])
    hf = _pool_gather(NP2, K2P)(h3, perm2_pad)
    out = _rowscale(K2P)(hf, jnp.pad(vals2, (0, K2P - K2)).reshape(K2P, 1))
    return out[:K2]
